# baseline (speedup 1.0000x reference)
import functools

import jax
import jax.numpy as jnp
from jax import lax
from jax.experimental import pallas as pl
from jax.experimental.pallas import tpu as pltpu
from jax.experimental.pallas import tpu_sc as plsc

DIM = 4096
NUM_EXPERTS = 64
TOP_K = 8
TOKENS = 16384
CAPACITY_FACTOR = 1.0

BLOCK_T = 1024
SUB_T = 256

NC = 2
NS = 16
NW = NC * NS
LANES = 16
TOK_PER_W = TOKENS // NW
GROUPS = TOK_PER_W // LANES


def _matmul_kernel(xa_ref, xb_ref, wt_ref, b_ref, logits_ref):
    wt = wt_ref[...]
    b = b_ref[...]
    half = BLOCK_T // 2
    for c in range(BLOCK_T // SUB_T):
        sl = pl.ds(c * SUB_T, SUB_T)
        if c < (BLOCK_T // SUB_T) // 2:
            x = xa_ref[pl.ds(c * SUB_T, SUB_T), :]
        else:
            x = xb_ref[pl.ds(c * SUB_T - half, SUB_T), :]
        logits_ref[sl, :] = lax.dot_general(
            x, wt, (((1,), (0,)), ((), ())), preferred_element_type=jnp.float32
        ) + b


def _tc_logits(x, wt, b2):
    grid = (TOKENS // BLOCK_T,)
    return pl.pallas_call(
        _matmul_kernel,
        grid=grid,
        in_specs=[
            pl.BlockSpec((BLOCK_T // 2, DIM), lambda i: (2 * i, 0)),
            pl.BlockSpec((BLOCK_T // 2, DIM), lambda i: (2 * i + 1, 0)),
            pl.BlockSpec((DIM, NUM_EXPERTS), lambda i: (0, 0)),
            pl.BlockSpec((1, NUM_EXPERTS), lambda i: (0, 0)),
        ],
        out_specs=pl.BlockSpec((BLOCK_T, NUM_EXPERTS), lambda i: (i, 0)),
        out_shape=jax.ShapeDtypeStruct((TOKENS, NUM_EXPERTS), jnp.float32),
        compiler_params=pltpu.CompilerParams(
            dimension_semantics=("parallel",),
        ),
    )(x, x, wt, b2)


def _sc_topk_body(logits_hbm, idx_hbm, wts_hbm, mask_hbm,
                  lg_v, idx_v, wts_v, mask_v):
    wid = lax.axis_index("s") * NC + lax.axis_index("c")
    base = wid * TOK_PER_W
    iota = lax.iota(jnp.int32, LANES)
    iota64 = iota * NUM_EXPERTS

    def group(g, carry):
        t0 = (base + g * LANES) * NUM_EXPERTS
        pltpu.sync_copy(logits_hbm.at[pl.ds(t0, LANES * NUM_EXPERTS)], lg_v)

        work = [plsc.load_gather(lg_v, [iota64 + e]) for e in range(NUM_EXPERTS)]
        ninf = jnp.full((LANES,), -jnp.inf, jnp.float32)
        vals = []
        idxs = []
        for _ in range(TOP_K):
            m = work[0]
            for e in range(1, NUM_EXPERTS):
                m = jnp.maximum(m, work[e])
            sel = jnp.full((LANES,), float(NUM_EXPERTS), jnp.float32)
            for e in range(NUM_EXPERTS):
                sel = jnp.minimum(
                    sel, jnp.where(work[e] == m, float(e), float(NUM_EXPERTS)))
            for e in range(NUM_EXPERTS):
                work[e] = jnp.where(sel == float(e), ninf, work[e])
            vals.append(m)
            idxs.append(sel)

        exps = [jnp.exp(v - vals[0]) for v in vals]
        s = exps[0]
        for k in range(1, TOP_K):
            s = s + exps[k]
        r = 1.0 / s

        iota8 = iota * TOP_K
        for k in range(TOP_K):
            plsc.store_scatter(idx_v, [iota8 + k], idxs[k].astype(jnp.int32))
            plsc.store_scatter(wts_v, [iota8 + k], exps[k] * r)
        one = jnp.ones((LANES,), jnp.float32)
        zero = jnp.zeros((LANES,), jnp.float32)
        for e in range(NUM_EXPERTS):
            plsc.store_scatter(mask_v, [iota64 + e],
                               jnp.where(work[e] == ninf, one, zero))

        tk0 = (base + g * LANES) * TOP_K
        pltpu.sync_copy(idx_v, idx_hbm.at[pl.ds(tk0, LANES * TOP_K)])
        pltpu.sync_copy(wts_v, wts_hbm.at[pl.ds(tk0, LANES * TOP_K)])
        pltpu.sync_copy(mask_v, mask_hbm.at[pl.ds(t0, LANES * NUM_EXPERTS)])
        return carry

    lax.fori_loop(0, GROUPS, group, 0)


def _sc_topk(logits):
    logits_flat = logits.reshape(-1)
    k = functools.partial(
        pl.kernel,
        mesh=plsc.VectorSubcoreMesh(core_axis_name="c", subcore_axis_name="s"),
        out_type=[
            jax.ShapeDtypeStruct((TOKENS * TOP_K,), jnp.int32),
            jax.ShapeDtypeStruct((TOKENS * TOP_K,), jnp.float32),
            jax.ShapeDtypeStruct((TOKENS * NUM_EXPERTS,), jnp.float32),
        ],
        scratch_types=[
            pltpu.VMEM((LANES * NUM_EXPERTS,), jnp.float32),
            pltpu.VMEM((LANES * TOP_K,), jnp.int32),
            pltpu.VMEM((LANES * TOP_K,), jnp.float32),
            pltpu.VMEM((LANES * NUM_EXPERTS,), jnp.float32),
        ],
        compiler_params=pltpu.CompilerParams(needs_layout_passes=False),
    )(_sc_topk_body)
    idx_f, wts_f, mask_f = k(logits_flat)
    return (idx_f.reshape(TOKENS, TOP_K),
            wts_f.reshape(TOKENS, TOP_K),
            mask_f.reshape(TOKENS, NUM_EXPERTS))


def kernel(x, W, b):
    wt = W.T
    b2 = b.reshape(1, NUM_EXPERTS)
    logits = _tc_logits(x, wt, b2)
    idx, wts, mask = _sc_topk(logits)
    capacity = min(TOKENS, int(CAPACITY_FACTOR * TOKENS / NUM_EXPERTS * TOP_K))
    return (logits, idx, wts, mask, jnp.int32(capacity))


# speedup vs baseline: 2.0981x; 2.0981x over previous
import jax
import jax.numpy as jnp
from jax import lax
from jax.experimental import pallas as pl
from jax.experimental.pallas import tpu as pltpu

DIM = 4096
NUM_EXPERTS = 64
TOP_K = 8
TOKENS = 16384
CAPACITY_FACTOR = 1.0

BLOCK_T = 1024


SUB_T = 256


def _router_kernel(xa_ref, xb_ref, wt_ref, b_ref, logits_ref, idx_ref, wts_ref,
                   mask_ref):
    wt = wt_ref[...]
    b = b_ref[...]
    half = BLOCK_T // 2
    for c in range(BLOCK_T // SUB_T):
        sl = pl.ds(c * SUB_T, SUB_T)
        if c < (BLOCK_T // SUB_T) // 2:
            x = xa_ref[pl.ds(c * SUB_T, SUB_T), :]
        else:
            x = xb_ref[pl.ds(c * SUB_T - half, SUB_T), :]
        logits = lax.dot_general(
            x, wt, (((1,), (0,)), ((), ())), preferred_element_type=jnp.float32
        ) + b
        logits_ref[sl, :] = logits

        iota_f = lax.broadcasted_iota(jnp.int32, logits.shape, 1).astype(jnp.float32)
        work = logits
        vals = []
        idxs = []
        for _ in range(TOP_K):
            m = jnp.max(work, axis=1, keepdims=True)
            cand = jnp.where(work == m, iota_f, float(NUM_EXPERTS))
            idx_f = jnp.min(cand, axis=1, keepdims=True)
            work = jnp.where(iota_f == idx_f, -jnp.inf, work)
            vals.append(m)
            idxs.append(idx_f)
        mask_ref[sl, :] = (work == -jnp.inf).astype(jnp.float32)

        tv = jnp.concatenate(vals, axis=1)
        ti = jnp.concatenate(idxs, axis=1)
        e = jnp.exp(tv - tv[:, 0:1])
        wts_ref[sl, :] = e / jnp.sum(e, axis=1, keepdims=True)
        idx_ref[sl, :] = ti.astype(jnp.int32)


def kernel(x, W, b):
    wt = W.T
    b2 = b.reshape(1, NUM_EXPERTS)
    grid = (TOKENS // BLOCK_T,)
    logits, idx, wts, mask = pl.pallas_call(
        _router_kernel,
        grid=grid,
        in_specs=[
            pl.BlockSpec((BLOCK_T // 2, DIM), lambda i: (2 * i, 0)),
            pl.BlockSpec((BLOCK_T // 2, DIM), lambda i: (2 * i + 1, 0)),
            pl.BlockSpec((DIM, NUM_EXPERTS), lambda i: (0, 0)),
            pl.BlockSpec((1, NUM_EXPERTS), lambda i: (0, 0)),
        ],
        out_specs=[
            pl.BlockSpec((BLOCK_T, NUM_EXPERTS), lambda i: (i, 0)),
            pl.BlockSpec((BLOCK_T, TOP_K), lambda i: (i, 0)),
            pl.BlockSpec((BLOCK_T, TOP_K), lambda i: (i, 0)),
            pl.BlockSpec((BLOCK_T, NUM_EXPERTS), lambda i: (i, 0)),
        ],
        out_shape=[
            jax.ShapeDtypeStruct((TOKENS, NUM_EXPERTS), jnp.float32),
            jax.ShapeDtypeStruct((TOKENS, TOP_K), jnp.int32),
            jax.ShapeDtypeStruct((TOKENS, TOP_K), jnp.float32),
            jax.ShapeDtypeStruct((TOKENS, NUM_EXPERTS), jnp.float32),
        ],
        compiler_params=pltpu.CompilerParams(
            dimension_semantics=("parallel",),
        ),
    )(x, x, wt, b2)
    capacity = min(TOKENS, int(CAPACITY_FACTOR * TOKENS / NUM_EXPERTS * TOP_K))
    return (logits, idx, wts, mask, jnp.int32(capacity))
